# baseline (speedup 1.0000x reference)
import jax
import jax.numpy as jnp
from jax.experimental import pallas as pl

_K = 8
_B = 64
_C = 100000
_PAD = 131072


def _make_gumbel():
    import numpy as np
    with jax.ensure_compile_time_eval():
        u = jax.random.uniform(jax.random.key(42), (_B, _C),
                               minval=1e-10, maxval=1.0)
        g = -jnp.log(-jnp.log(u))
    return np.asarray(g)


_GUM = _make_gumbel()


def _eighth_largest_exact(p):
    t = p
    for _ in range(_K - 1):
        m = jnp.max(t, axis=-1, keepdims=True)
        t = jnp.where(t >= m, -jnp.inf, t)
    return jnp.max(t, axis=-1, keepdims=True)


def _body(s_ref, g_ref, o_ref):
    neg = jnp.float32(-jnp.inf)
    p = s_ref[...] + g_ref[...]
    pad = jnp.full((8, _PAD - _C), neg, jnp.float32)
    x = jnp.concatenate([p, pad], axis=1)
    h = _PAD // 2
    m1 = jnp.maximum(x[:, :h], x[:, h:])
    m2 = jnp.minimum(x[:, :h], x[:, h:])
    w = h
    while w > 1024:
        h = w // 2
        a1, a2 = m1[:, :h], m1[:, h:w]
        b1, b2 = m2[:, :h], m2[:, h:w]
        m1 = jnp.maximum(a1, a2)
        m2 = jnp.maximum(jnp.minimum(a1, a2), jnp.maximum(b1, b2))
        w = h
    cand = jnp.concatenate([m1, m2], axis=1)
    t8 = _eighth_largest_exact(cand)
    o = jnp.where(p >= t8, 1.0, 0.0).astype(jnp.float32)
    cnt = jnp.sum(o, axis=-1, keepdims=True)
    ok = jnp.all(cnt == float(_K))
    o_ref[...] = o[None]

    @pl.when(jnp.logical_not(ok))
    def _fallback():
        p2 = s_ref[...] + g_ref[...]
        thr = _eighth_largest_exact(p2)
        o_ref[...] = jnp.where(p2 >= thr, 1.0, 0.0).astype(jnp.float32)[None]


def kernel(scores):
    s = scores.reshape(_B, _C)
    g = _GUM
    out = pl.pallas_call(
        _body,
        grid=(8,),
        in_specs=[
            pl.BlockSpec((8, _C), lambda i: (i, 0)),
            pl.BlockSpec((8, _C), lambda i: (i, 0)),
        ],
        out_specs=pl.BlockSpec((1, 8, _C), lambda i: (0, i, 0)),
        out_shape=jax.ShapeDtypeStruct((1, _B, _C), jnp.float32),
    )(s, g)
    return out


# speedup vs baseline: 1.0187x; 1.0187x over previous
import numpy as np
import jax
import jax.numpy as jnp
from jax.experimental import pallas as pl

_K = 8
_B = 64
_C = 100000
_NFULL = _C // 128
_REM = _C - _NFULL * 128


def _make_gumbel():
    def expr():
        u = jax.random.uniform(jax.random.key(42), (_B, _C),
                               minval=1e-10, maxval=1.0)
        return -jnp.log(-jnp.log(u))
    return np.asarray(jax.jit(expr)())


_GUM = _make_gumbel()


def _eighth_largest_exact(p):
    t = p
    for _ in range(_K - 1):
        m = jnp.max(t, axis=-1, keepdims=True)
        t = jnp.where(t >= m, -jnp.inf, t)
    return jnp.max(t, axis=-1, keepdims=True)


def _chunk(s_ref, g_ref, j):
    if j < _NFULL:
        sl = slice(j * 128, (j + 1) * 128)
        return s_ref[:, sl] + g_ref[:, sl]
    sl = slice(_NFULL * 128, _C)
    x = s_ref[:, sl] + g_ref[:, sl]
    return jnp.concatenate(
        [x, jnp.full((8, 128 - _REM), -jnp.inf, jnp.float32)], axis=1)


def _body(s_ref, g_ref, o_ref):
    neg = jnp.float32(-jnp.inf)
    m1 = jnp.full((8, 128), neg, jnp.float32)
    m2 = m1
    m3 = m1
    for j in range(_NFULL + 1):
        x = _chunk(s_ref, g_ref, j)
        hi1 = jnp.maximum(m1, x)
        lo1 = jnp.minimum(m1, x)
        m1 = hi1
        hi2 = jnp.maximum(m2, lo1)
        lo2 = jnp.minimum(m2, lo1)
        m2 = hi2
        m3 = jnp.maximum(m3, lo2)
    cand = jnp.concatenate([m1, m2, m3], axis=1)
    t8 = _eighth_largest_exact(cand)

    cnt = jnp.zeros((8, 128), jnp.float32)
    for j in range(_NFULL + 1):
        x = _chunk(s_ref, g_ref, j)
        o = jnp.where(x >= t8, 1.0, 0.0).astype(jnp.float32)
        if j < _NFULL:
            o_ref[0, :, j * 128:(j + 1) * 128] = o
            cnt = cnt + o
        else:
            o_ref[0, :, _NFULL * 128:_C] = o[:, :_REM]
            cnt = cnt + o
    ok = jnp.all(jnp.sum(cnt, axis=-1) == float(_K))

    @pl.when(jnp.logical_not(ok))
    def _fallback():
        p = s_ref[...] + g_ref[...]
        thr = _eighth_largest_exact(p)
        o_ref[...] = jnp.where(p >= thr, 1.0, 0.0).astype(jnp.float32)[None]


def kernel(scores):
    s = scores.reshape(_B, _C)
    g = _GUM
    out = pl.pallas_call(
        _body,
        grid=(8,),
        in_specs=[
            pl.BlockSpec((8, _C), lambda i: (i, 0)),
            pl.BlockSpec((8, _C), lambda i: (i, 0)),
        ],
        out_specs=pl.BlockSpec((1, 8, _C), lambda i: (0, i, 0)),
        out_shape=jax.ShapeDtypeStruct((1, _B, _C), jnp.float32),
    )(s, g)
    return out
